# Initial kernel scaffold; baseline (speedup 1.0000x reference)
import jax
import jax.numpy as jnp
from jax.experimental import pallas as pl


def kernel(x, edge_index, edge_attr, W1a, b1a, W1b, b1b, eps1, W2a, b2a, W2b, b2b, eps2, Wl, bl, Wr):
    raise NotImplementedError("write your pallas kernel here")


# speedup vs baseline: 4.0936x; 4.0936x over previous
import jax
import jax.numpy as jnp
from jax import lax
from jax.experimental import pallas as pl
from jax.experimental.pallas import tpu as pltpu
from jax.experimental.pallas import tpu_sc as plsc

N = 10000
E = 320000
F = 128
H = 16
H2 = 8

NC = 2
NS = 16
L = 16
NW = NC * NS
CHUNK = 128
CPW = 80
C_TOT = NW * CPW
E_PAD = C_TOT * CHUNK
N_PAD = 10240
RPT = N_PAD // NS

_MESH = dict(core_axis_name="c", subcore_axis_name="s", num_cores=NC,
             num_subcores=NS)
_SC_PARAMS = pltpu.CompilerParams(use_tc_tiling_on_sc=False)


FH = F // 2


def _seg64_body(src_hbm, dst_hbm, tab_hbm, out_hbm, deg_hbm,
                idx_s, idx_d, rows, ones, zb, zb1, cp, cpd, acc, accd, sem):
    cid = lax.axis_index("c")
    sid = lax.axis_index("s")
    wid = cid * NS + sid
    base = sid * RPT

    for r in range(16):
        for q in range(FH // L):
            zb[r, pl.ds(q * L, L)] = jnp.zeros((L,), jnp.float32)
    zb1[...] = jnp.zeros((L,), jnp.float32)
    for q in range(CHUNK // L):
        ones[pl.ds(q * L, L)] = jnp.ones((L,), jnp.float32)

    def zloop(i, c):
        pltpu.sync_copy(zb, acc.at[pl.ds(base + i * 16, 16)])
        pltpu.sync_copy(zb1, accd.at[pl.ds(base + i * 16, 16)])
        return c
    lax.fori_loop(0, RPT // 16, zloop, 0)

    pltpu.sync_copy(src_hbm.at[pl.ds(wid * CPW, CPW)], idx_s)
    pltpu.sync_copy(dst_hbm.at[pl.ds(wid * CPW, CPW)], idx_d)
    plsc.subcore_barrier()

    def body(j, c):
        pltpu.async_copy(tab_hbm.at[idx_s.at[j]], rows, sem).wait()
        pltpu.sync_copy(rows, acc.at[idx_d.at[j]], add=True)
        pltpu.sync_copy(ones, accd.at[idx_d.at[j]], add=True)
        return c
    lax.fori_loop(0, CPW, body, 0)
    plsc.subcore_barrier()

    for t in range(2):
        pltpu.sync_copy(acc.at[pl.ds(base + t * (RPT // 2), RPT // 2)], cp)
        pltpu.sync_copy(cp, out_hbm.at[cid,
                                       pl.ds(base + t * (RPT // 2), RPT // 2)])
    pltpu.sync_copy(accd.at[pl.ds(base, RPT)], cpd)
    pltpu.sync_copy(cpd, deg_hbm.at[cid, pl.ds(base, RPT)])


def _seg64(src2d, dst2d, table):
    mesh = plsc.VectorSubcoreMesh(**_MESH)
    k = pl.kernel(
        _seg64_body,
        out_type=[jax.ShapeDtypeStruct((NC, N_PAD, FH), jnp.float32),
                  jax.ShapeDtypeStruct((NC, N_PAD), jnp.float32)],
        mesh=mesh,
        compiler_params=_SC_PARAMS,
        scratch_types=[
            pltpu.VMEM((CPW, CHUNK), jnp.int32),
            pltpu.VMEM((CPW, CHUNK), jnp.int32),
            pltpu.VMEM((CHUNK, FH), jnp.float32),
            pltpu.VMEM((CHUNK,), jnp.float32),
            pltpu.VMEM((16, FH), jnp.float32),
            pltpu.VMEM((L,), jnp.float32),
            pltpu.VMEM((RPT // 2, FH), jnp.float32),
            pltpu.VMEM((RPT,), jnp.float32),
            pltpu.VMEM_SHARED((N_PAD, FH), jnp.float32),
            pltpu.VMEM_SHARED((N_PAD,), jnp.float32),
            pltpu.SemaphoreType.DMA,
        ],
    )
    return k(src2d, dst2d, table)


def _seg16_body(src_hbm, dst_hbm, tab_hbm, out_hbm,
                idx_s, idx_d, rows, zb, cp, acc, sem):
    cid = lax.axis_index("c")
    sid = lax.axis_index("s")
    wid = cid * NS + sid
    base = sid * RPT

    for r in range(16):
        zb[r] = jnp.zeros((L,), jnp.float32)

    def zloop(i, c):
        pltpu.sync_copy(zb, acc.at[pl.ds(base + i * 16, 16)])
        return c
    lax.fori_loop(0, RPT // 16, zloop, 0)

    pltpu.sync_copy(src_hbm.at[pl.ds(wid * CPW, CPW)], idx_s)
    pltpu.sync_copy(dst_hbm.at[pl.ds(wid * CPW, CPW)], idx_d)
    plsc.subcore_barrier()

    def body(j, c):
        pltpu.async_copy(tab_hbm.at[idx_s.at[j]], rows, sem).wait()
        pltpu.sync_copy(rows, acc.at[idx_d.at[j]], add=True)
        return c
    lax.fori_loop(0, CPW, body, 0)
    plsc.subcore_barrier()

    pltpu.sync_copy(acc.at[pl.ds(base, RPT)], cp)
    pltpu.sync_copy(cp, out_hbm.at[cid, pl.ds(base, RPT)])


def _seg16(src2d, dst2d, table):
    mesh = plsc.VectorSubcoreMesh(**_MESH)
    k = pl.kernel(
        _seg16_body,
        out_type=jax.ShapeDtypeStruct((NC, N_PAD, H), jnp.float32),
        mesh=mesh,
        compiler_params=_SC_PARAMS,
        scratch_types=[
            pltpu.VMEM((CPW, CHUNK), jnp.int32),
            pltpu.VMEM((CPW, CHUNK), jnp.int32),
            pltpu.VMEM((CHUNK, H), jnp.float32),
            pltpu.VMEM((16, H), jnp.float32),
            pltpu.VMEM((RPT, H), jnp.float32),
            pltpu.VMEM_SHARED((N_PAD, H), jnp.float32),
            pltpu.SemaphoreType.DMA,
        ],
    )
    return k(src2d, dst2d, table)


def _segw_body(src_hbm, dst_hbm, w_hbm, tab_hbm, out_hbm,
               idx_s, idx_d, rows, wbuf, zb, cp, acc, sem):
    cid = lax.axis_index("c")
    sid = lax.axis_index("s")
    wid = cid * NS + sid
    base = sid * RPT

    for r in range(16):
        zb[r] = jnp.zeros((L,), jnp.float32)

    def zloop(i, c):
        pltpu.sync_copy(zb, acc.at[pl.ds(base + i * 16, 16)])
        return c
    lax.fori_loop(0, RPT // 16, zloop, 0)

    pltpu.sync_copy(src_hbm.at[pl.ds(wid * CPW, CPW)], idx_s)
    pltpu.sync_copy(dst_hbm.at[pl.ds(wid * CPW, CPW)], idx_d)
    plsc.subcore_barrier()

    def body(j, c):
        pltpu.sync_copy(w_hbm.at[wid * CPW + j], wbuf)
        pltpu.async_copy(tab_hbm.at[idx_s.at[j]], rows, sem).wait()
        for r in range(CHUNK):
            rows[r] = rows[r] * wbuf[r]
        pltpu.sync_copy(rows, acc.at[idx_d.at[j]], add=True)
        return c
    lax.fori_loop(0, CPW, body, 0)
    plsc.subcore_barrier()

    pltpu.sync_copy(acc.at[pl.ds(base, RPT)], cp)
    pltpu.sync_copy(cp, out_hbm.at[cid, pl.ds(base, RPT)])


def _segw(src2d, dst2d, w3d, table):
    mesh = plsc.VectorSubcoreMesh(**_MESH)
    k = pl.kernel(
        _segw_body,
        out_type=jax.ShapeDtypeStruct((NC, N_PAD, H), jnp.float32),
        mesh=mesh,
        compiler_params=_SC_PARAMS,
        scratch_types=[
            pltpu.VMEM((CPW, CHUNK), jnp.int32),
            pltpu.VMEM((CPW, CHUNK), jnp.int32),
            pltpu.VMEM((CHUNK, H), jnp.float32),
            pltpu.VMEM((CHUNK, H), jnp.float32),
            pltpu.VMEM((16, H), jnp.float32),
            pltpu.VMEM((RPT, H), jnp.float32),
            pltpu.VMEM_SHARED((N_PAD, H), jnp.float32),
            pltpu.SemaphoreType.DMA,
        ],
    )
    return k(src2d, dst2d, w3d, table)


_BS = 2000


def _gin_layer(scale, xin, accs, Wa, ba, Wb, bb, fin, fout):
    na = len(accs)

    def body(*refs):
        s_ref = refs[0]
        x_ref = refs[1]
        a_refs = refs[2:2 + na]
        wa_ref, ba_ref, wb_ref, bb_ref, o_ref = refs[2 + na:]
        agg = jnp.concatenate([a[0] + a[1] for a in a_refs], axis=1)
        z = s_ref[0, 0] * x_ref[...] + agg
        t = jnp.dot(z, wa_ref[...],
                    preferred_element_type=jnp.float32) + ba_ref[...]
        t = jnp.dot(t, wb_ref[...],
                    preferred_element_type=jnp.float32) + bb_ref[...]
        o_ref[...] = jnp.maximum(t, 0.0)
    return pl.pallas_call(
        body,
        grid=(N // _BS,),
        in_specs=[
            pl.BlockSpec((1, 1), lambda i: (0, 0), memory_space=pltpu.SMEM),
            pl.BlockSpec((_BS, fin), lambda i: (i, 0)),
        ] + [
            pl.BlockSpec((NC, _BS, a.shape[2]), lambda i: (0, i, 0))
            for a in accs
        ] + [
            pl.BlockSpec((fin, fout), lambda i: (0, 0)),
            pl.BlockSpec((1, fout), lambda i: (0, 0)),
            pl.BlockSpec((fout, fout), lambda i: (0, 0)),
            pl.BlockSpec((1, fout), lambda i: (0, 0)),
        ],
        out_specs=pl.BlockSpec((_BS, fout), lambda i: (i, 0)),
        out_shape=jax.ShapeDtypeStruct((N, fout), jnp.float32),
    )(scale, xin, *accs, Wa, ba, Wb, bb)


def _pad16(h2):
    def body(h_ref, o_ref):
        o_ref[...] = jnp.concatenate(
            [h_ref[...], jnp.zeros((_BS, H - H2), jnp.float32)], axis=1)
    return pl.pallas_call(
        body,
        grid=(N // _BS,),
        in_specs=[pl.BlockSpec((_BS, H2), lambda i: (i, 0))],
        out_specs=pl.BlockSpec((_BS, H), lambda i: (i, 0)),
        out_shape=jax.ShapeDtypeStruct((N, H), jnp.float32),
    )(h2)


def _sage_out(acc3, degp, h2, Wl, bl, Wr):
    def body(a_ref, d_ref, h_ref, wl_ref, bl_ref, wr_ref, o_ref):
        s = a_ref[0, :, :H2] + a_ref[1, :, :H2]
        deg = d_ref[0] + d_ref[1]
        mean = s / jnp.maximum(deg, 1.0)
        o_ref[...] = jnp.maximum(
            jnp.dot(mean, wl_ref[...], preferred_element_type=jnp.float32)
            + bl_ref[0, 0]
            + jnp.dot(h_ref[...], wr_ref[...],
                      preferred_element_type=jnp.float32), 0.0)
    return pl.pallas_call(
        body,
        grid=(N // _BS,),
        in_specs=[
            pl.BlockSpec((NC, _BS, H), lambda i: (0, i, 0)),
            pl.BlockSpec((NC, _BS, 1), lambda i: (0, i, 0)),
            pl.BlockSpec((_BS, H2), lambda i: (i, 0)),
            pl.BlockSpec((H2, 1), lambda i: (0, 0)),
            pl.BlockSpec((1, 1), lambda i: (0, 0), memory_space=pltpu.SMEM),
            pl.BlockSpec((H2, 1), lambda i: (0, 0)),
        ],
        out_specs=pl.BlockSpec((_BS, 1), lambda i: (i, 0)),
        out_shape=jax.ShapeDtypeStruct((N, 1), jnp.float32),
    )(acc3, degp, h2, Wl, bl, Wr)


def kernel(x, edge_index, edge_attr, W1a, b1a, W1b, b1b, eps1,
           W2a, b2a, W2b, b2b, eps2, Wl, bl, Wr):
    pad = E_PAD - E
    src = jnp.concatenate(
        [edge_index[0], jnp.zeros((pad,), jnp.int32)]).reshape(C_TOT, CHUNK)
    dst = jnp.concatenate(
        [edge_index[1], jnp.full((pad,), N, jnp.int32)]).reshape(C_TOT, CHUNK)
    w_pad = jnp.concatenate(
        [edge_attr.reshape(-1), jnp.zeros((pad,), jnp.float32)])
    w3d = jnp.broadcast_to(w_pad[:, None],
                           (E_PAD, H)).reshape(C_TOT, CHUNK, H)
    scale1 = (1.0 + eps1).reshape(1, 1)
    scale2 = (1.0 + eps2).reshape(1, 1)

    accL, degp = _seg64(src, dst, x[:, :FH])
    accR, _ = _seg64(src, dst, x[:, FH:])
    h1 = _gin_layer(scale1, x, [accL, accR], W1a, b1a.reshape(1, H), W1b,
                    b1b.reshape(1, H), F, H)
    acc2 = _seg16(src, dst, h1)
    h2 = _gin_layer(scale2, h1, [acc2], W2a, b2a.reshape(1, H2), W2b,
                    b2b.reshape(1, H2), H, H2)
    t3 = _pad16(h2)
    acc3 = _segw(src, dst, w3d, t3)
    return _sage_out(acc3, degp[:, :, None], h2, Wl, bl.reshape(1, 1), Wr)
